# baseline (speedup 1.0000x reference)
import functools

import jax
import jax.numpy as jnp
from jax import lax
from jax.experimental import pallas as pl
from jax.experimental.pallas import tpu as pltpu
from jax.experimental.pallas import tpu_sc as plsc

_K = 128


_N_SLOW = 32
_N_FAST = 128


@functools.lru_cache(maxsize=None)
def _make_segsum(N, E, D):
    info = plsc.get_sparse_core_info()
    NC, NS = info.num_cores, info.num_subcores
    n0, n1 = _N_SLOW, _N_FAST
    assert NS * (n0 + n1) * _K >= E
    chunks = max(n0, n1)
    calloc = NS * (n0 + n1)
    rows_pw = -(-(N + 1) // (NS * 8)) * 8
    NPAD = rows_pw * NS

    mesh = plsc.VectorSubcoreMesh(core_axis_name="c", subcore_axis_name="s")

    @functools.partial(
        pl.kernel,
        mesh=mesh,
        out_type=jax.ShapeDtypeStruct((NC, NPAD, D), jnp.float32),
        scratch_types=[
            pltpu.VMEM((chunks, _K), jnp.int32),
            pltpu.VMEM((_K,), jnp.int32),
            pltpu.VMEM((_K,), jnp.int32),
            pltpu.VMEM((_K, D), jnp.float32),
            pltpu.VMEM((_K, D), jnp.float32),
            pltpu.VMEM_SHARED((NPAD, D), jnp.float32),
            pltpu.SemaphoreType.DMA,
            pltpu.SemaphoreType.DMA,
            pltpu.SemaphoreType.DMA,
            pltpu.SemaphoreType.DMA,
        ],
    )
    def segsum(h_hbm, src_hbm, dst_hbm, zeros_hbm, out_hbm,
               sidx, db0, db1, rows0, rows1, acc, gs0, gs1, is0, is1):
        cid = lax.axis_index("c")
        sid = lax.axis_index("s")
        pltpu.sync_copy(zeros_hbm.at[pl.ds(sid * rows_pw, rows_pw), :],
                        acc.at[pl.ds(sid * rows_pw, rows_pw), :])
        plsc.subcore_barrier()

        def pipeline(base_row, count):
            prow = pl.multiple_of(base_row, 8)
            pltpu.sync_copy(src_hbm.at[pl.ds(prow, count), :],
                            sidx.at[pl.ds(0, count), :])

            def gather(l, buf, sem):
                return pltpu.make_async_copy(h_hbm.at[sidx.at[l]], buf, sem)

            def dloadc(l, buf, sem):
                return pltpu.make_async_copy(dst_hbm.at[prow + l], buf, sem)

            gather(0, rows0, gs0).start()
            dloadc(0, db0, is0).start()
            gather(1, rows1, gs1).start()
            dloadc(1, db1, is1).start()

            def body(j, carry):
                l0 = j * 2
                gather(l0, rows0, gs0).wait()
                dloadc(l0, db0, is0).wait()
                pltpu.sync_copy(rows0, acc.at[db0], add=True)

                @pl.when(j < count // 2 - 1)
                def _():
                    gather(l0 + 2, rows0, gs0).start()
                    dloadc(l0 + 2, db0, is0).start()

                gather(l0 + 1, rows1, gs1).wait()
                dloadc(l0 + 1, db1, is1).wait()
                pltpu.sync_copy(rows1, acc.at[db1], add=True)

                @pl.when(j < count // 2 - 1)
                def _():
                    gather(l0 + 3, rows1, gs1).start()
                    dloadc(l0 + 3, db1, is1).start()

                return carry

            lax.fori_loop(0, count // 2, body, 0)

        @pl.when(cid == 0)
        def _():
            pipeline(sid * n0, n0)

        @pl.when(cid == 1)
        def _():
            pipeline(NS * n0 + sid * n1, n1)

        plsc.subcore_barrier()
        pltpu.sync_copy(acc.at[pl.ds(sid * rows_pw, rows_pw), :],
                        out_hbm.at[cid, pl.ds(sid * rows_pw, rows_pw), :])

    return segsum, NPAD, calloc


def _bn(x, g, b):
    m = jnp.mean(x, axis=0, keepdims=True)
    v = jnp.mean((x - m) ** 2, axis=0, keepdims=True)
    return (x - m) / jnp.sqrt(v + 1e-5) * g + b


def _emb_body(h_ref, We_ref, be_ref, out_ref):
    out_ref[...] = jnp.dot(h_ref[...], We_ref[...],
                           preferred_element_type=jnp.float32) + be_ref[...]


def _mlp_body(h_ref, parts_ref, W1_ref, b1_ref, g1_ref, bt1_ref,
              W2_ref, b2_ref, ag_ref, ab_ref, lg_ref, lb_ref, out_ref):
    h = h_ref[...]
    n = h.shape[0]
    z = h + parts_ref[0, :n] + parts_ref[1, :n]
    u = jnp.dot(z, W1_ref[...], preferred_element_type=jnp.float32) + b1_ref[...]
    t = jnp.maximum(_bn(u, g1_ref[...], bt1_ref[...]), 0.0)
    t = jnp.dot(t, W2_ref[...], preferred_element_type=jnp.float32) + b2_ref[...]
    t = jnp.maximum(_bn(t, ag_ref[...], ab_ref[...]), 0.0)
    t = _bn(t, lg_ref[...], lb_ref[...])
    t = jnp.maximum(t, 0.0)
    out_ref[...] = h + t


@functools.lru_cache(maxsize=None)
def _make_dense(N, D):
    emb = pl.pallas_call(
        _emb_body, out_shape=jax.ShapeDtypeStruct((N, D), jnp.float32))
    mlp = pl.pallas_call(
        _mlp_body, out_shape=jax.ShapeDtypeStruct((N, D), jnp.float32))
    return emb, mlp


def kernel(h, edge_index, e, We, be, mW1, mb1, mg1, mbt1, mW2, mb2,
           ag, ab, lg, lb):
    N, D = h.shape
    E = edge_index.shape[1]
    L = mW1.shape[0]
    segsum, NPAD, calloc = _make_segsum(N, E, D)
    emb, mlp = _make_dense(N, D)

    src = edge_index[0].astype(jnp.int32)
    dst = edge_index[1].astype(jnp.int32)
    pad = calloc * _K - E
    if pad:
        src = jnp.concatenate([src, jnp.zeros((pad,), jnp.int32)])
        dst = jnp.concatenate([dst, jnp.full((pad,), N, jnp.int32)])
    src = src.reshape(calloc, _K)
    dst = dst.reshape(calloc, _K)
    zeros = jnp.zeros((NPAD, D), jnp.float32)

    r1 = lambda a: a.reshape(1, D)
    h = emb(h, We, r1(be))
    for l in range(L):
        parts = segsum(h, src, dst, zeros)
        h = mlp(h, parts, mW1[l], r1(mb1[l]), r1(mg1[l]), r1(mbt1[l]),
                mW2[l], r1(mb2[l]), r1(ag[l]), r1(ab[l]), r1(lg[l]), r1(lb[l]))
    return h


# speedup vs baseline: 1.1027x; 1.1027x over previous
import functools

import jax
import jax.numpy as jnp
from jax import lax
from jax.experimental import pallas as pl
from jax.experimental.pallas import tpu as pltpu
from jax.experimental.pallas import tpu_sc as plsc

_K = 128


_ACTIVE_CID = 0


@functools.lru_cache(maxsize=None)
def _make_segsum(N, E, D):
    info = plsc.get_sparse_core_info()
    NC, NS = info.num_cores, info.num_subcores
    n = -(-E // (_K * NS * 4)) * 4
    calloc = NS * n
    rows_pw = -(-(N + 1) // (NS * 8)) * 8
    NPAD = rows_pw * NS

    mesh = plsc.VectorSubcoreMesh(core_axis_name="c", subcore_axis_name="s")

    @functools.partial(
        pl.kernel,
        mesh=mesh,
        out_type=jax.ShapeDtypeStruct((NPAD, D), jnp.float32),
        scratch_types=(
            [pltpu.VMEM((_K,), jnp.int32)] * 8 +
            [pltpu.VMEM((_K, D), jnp.float32)] * 2 +
            [pltpu.VMEM_SHARED((NPAD, D), jnp.float32)] +
            [pltpu.SemaphoreType.DMA] * 6
        ),
    )
    def segsum(h_hbm, src_hbm, dst_hbm, zeros_hbm, out_hbm,
               sb0, sb1, sb2, sb3, db0, db1, db2, db3, rows0, rows1, acc,
               gs0, gs1, is0, is1, is2, is3):
        cid = lax.axis_index("c")
        sid = lax.axis_index("s")
        sb = (sb0, sb1, sb2, sb3)
        db = (db0, db1, db2, db3)
        isem = (is0, is1, is2, is3)
        rbuf = (rows0, rows1)
        gsem = (gs0, gs1)

        @pl.when(cid == _ACTIVE_CID)
        def _work():
            row0 = sid * n
            pltpu.sync_copy(zeros_hbm.at[pl.ds(sid * rows_pw, rows_pw), :],
                            acc.at[pl.ds(sid * rows_pw, rows_pw), :])
            plsc.subcore_barrier()

            def iload(c, k):
                a = pltpu.make_async_copy(src_hbm.at[row0 + c], sb[k],
                                          isem[k])
                b = pltpu.make_async_copy(dst_hbm.at[row0 + c], db[k],
                                          isem[k])
                return a, b

            def istart(c, k):
                a, b = iload(c, k)
                a.start()
                b.start()

            def iwait(c, k):
                a, b = iload(c, k)
                a.wait()
                b.wait()

            def gather(c, k, r):
                return pltpu.make_async_copy(h_hbm.at[sb[k]], rbuf[r],
                                             gsem[r])

            for k in range(4):
                istart(k, k)
            iwait(0, 0)
            gather(0, 0, 0).start()
            iwait(1, 1)
            gather(1, 1, 1).start()

            def chunk(c, k):
                r = k % 2
                gather(c, k, r).wait()
                pltpu.sync_copy(rbuf[r], acc.at[db[k]], add=True)

                @pl.when(c + 2 < n)
                def _():
                    k2 = (k + 2) % 4
                    iwait(c + 2, k2)
                    gather(c + 2, k2, r).start()

                @pl.when(c + 4 < n)
                def _():
                    istart(c + 4, k)

            def body(j, carry):
                c0 = j * 4
                for k in range(4):
                    chunk(c0 + k, k)
                return carry

            lax.fori_loop(0, n // 4, body, 0)
            plsc.subcore_barrier()
            pltpu.sync_copy(acc.at[pl.ds(sid * rows_pw, rows_pw), :],
                            out_hbm.at[pl.ds(sid * rows_pw, rows_pw), :])

    return segsum, NPAD, calloc


def _bn(x, g, b):
    m = jnp.mean(x, axis=0, keepdims=True)
    v = jnp.mean((x - m) ** 2, axis=0, keepdims=True)
    return (x - m) / jnp.sqrt(v + 1e-5) * g + b


def _emb_body(h_ref, We_ref, be_ref, out_ref):
    out_ref[...] = jnp.dot(h_ref[...], We_ref[...],
                           preferred_element_type=jnp.float32) + be_ref[...]


def _mlp_body(h_ref, parts_ref, W1_ref, b1_ref, g1_ref, bt1_ref,
              W2_ref, b2_ref, ag_ref, ab_ref, lg_ref, lb_ref, out_ref):
    h = h_ref[...]
    n = h.shape[0]
    z = h + parts_ref[:n]
    u = jnp.dot(z, W1_ref[...], preferred_element_type=jnp.float32) + b1_ref[...]
    t = jnp.maximum(_bn(u, g1_ref[...], bt1_ref[...]), 0.0)
    t = jnp.dot(t, W2_ref[...], preferred_element_type=jnp.float32) + b2_ref[...]
    t = jnp.maximum(_bn(t, ag_ref[...], ab_ref[...]), 0.0)
    t = _bn(t, lg_ref[...], lb_ref[...])
    t = jnp.maximum(t, 0.0)
    out_ref[...] = h + t


@functools.lru_cache(maxsize=None)
def _make_dense(N, D):
    emb = pl.pallas_call(
        _emb_body, out_shape=jax.ShapeDtypeStruct((N, D), jnp.float32))
    mlp = pl.pallas_call(
        _mlp_body, out_shape=jax.ShapeDtypeStruct((N, D), jnp.float32))
    return emb, mlp


def kernel(h, edge_index, e, We, be, mW1, mb1, mg1, mbt1, mW2, mb2,
           ag, ab, lg, lb):
    N, D = h.shape
    E = edge_index.shape[1]
    L = mW1.shape[0]
    segsum, NPAD, calloc = _make_segsum(N, E, D)
    emb, mlp = _make_dense(N, D)

    src = edge_index[0].astype(jnp.int32)
    dst = edge_index[1].astype(jnp.int32)
    pad = calloc * _K - E
    if pad:
        src = jnp.concatenate([src, jnp.zeros((pad,), jnp.int32)])
        dst = jnp.concatenate([dst, jnp.full((pad,), N, jnp.int32)])
    src = src.reshape(calloc, _K)
    dst = dst.reshape(calloc, _K)
    zeros = jnp.zeros((NPAD, D), jnp.float32)

    r1 = lambda a: a.reshape(1, D)
    h = emb(h, We, r1(be))
    for l in range(L):
        parts = segsum(h, src, dst, zeros)
        h = mlp(h, parts, mW1[l], r1(mb1[l]), r1(mg1[l]), r1(mbt1[l]),
                mW2[l], r1(mb2[l]), r1(ag[l]), r1(ab[l]), r1(lg[l]), r1(lb[l]))
    return h


# speedup vs baseline: 1.3665x; 1.2392x over previous
import functools

import jax
import jax.numpy as jnp
from jax import lax
from jax.experimental import pallas as pl
from jax.experimental.pallas import tpu as pltpu
from jax.experimental.pallas import tpu_sc as plsc

_K = 128


_ACTIVE_CID = 0


@functools.lru_cache(maxsize=None)
def _make_segsum(N, E, D):
    info = plsc.get_sparse_core_info()
    NC, NS = info.num_cores, info.num_subcores
    n = -(-E // (_K * NS))
    n += n % 2
    nloops = n // 2
    calloc = NS * n
    rows_pw = -(-(N + 1) // (NS * 8)) * 8
    NPAD = rows_pw * NS

    mesh = plsc.VectorSubcoreMesh(core_axis_name="c", subcore_axis_name="s")

    @functools.partial(
        pl.kernel,
        mesh=mesh,
        out_type=jax.ShapeDtypeStruct((NPAD, D), jnp.float32),
        scratch_types=[
            pltpu.VMEM((_K,), jnp.int32),
            pltpu.VMEM((_K,), jnp.int32),
            pltpu.VMEM((_K,), jnp.int32),
            pltpu.VMEM((_K,), jnp.int32),
            pltpu.VMEM((_K, D), jnp.float32),
            pltpu.VMEM((_K, D), jnp.float32),
            pltpu.VMEM_SHARED((NPAD, D), jnp.float32),
            pltpu.SemaphoreType.DMA,
            pltpu.SemaphoreType.DMA,
            pltpu.SemaphoreType.DMA,
            pltpu.SemaphoreType.DMA,
        ],
    )
    def segsum(h_hbm, src_hbm, dst_hbm, zeros_hbm, out_hbm,
               sb0, sb1, db0, db1, rows0, rows1, acc, gs0, gs1, is0, is1):
        cid = lax.axis_index("c")
        sid = lax.axis_index("s")

        @pl.when(cid == _ACTIVE_CID)
        def _work():
            row0 = sid * n
            pltpu.sync_copy(zeros_hbm.at[pl.ds(sid * rows_pw, rows_pw), :],
                            acc.at[pl.ds(sid * rows_pw, rows_pw), :])

            def iload(c, sb, db, sem):
                a = pltpu.make_async_copy(src_hbm.at[row0 + c], sb, sem)
                b = pltpu.make_async_copy(dst_hbm.at[row0 + c], db, sem)
                return a, b

            def istart(c, sb, db, sem):
                a, b = iload(c, sb, db, sem)
                a.start()
                b.start()

            def iwait(c, sb, db, sem):
                a, b = iload(c, sb, db, sem)
                a.wait()
                b.wait()

            def gather(c, sb, buf, sem):
                return pltpu.make_async_copy(h_hbm.at[sb], buf, sem)

            plsc.subcore_barrier()
            istart(0, sb0, db0, is0)
            istart(1, sb1, db1, is1)
            iwait(0, sb0, db0, is0)
            gather(0, sb0, rows0, gs0).start()

            def body(j, carry):
                c0 = j * 2
                iwait(c0 + 1, sb1, db1, is1)
                gather(c0 + 1, sb1, rows1, gs1).start()
                gather(c0, sb0, rows0, gs0).wait()
                pltpu.sync_copy(rows0, acc.at[db0], add=True)

                @pl.when(j < nloops - 1)
                def _():
                    istart(c0 + 2, sb0, db0, is0)

                gather(c0 + 1, sb1, rows1, gs1).wait()
                pltpu.sync_copy(rows1, acc.at[db1], add=True)

                @pl.when(j < nloops - 1)
                def _():
                    istart(c0 + 3, sb1, db1, is1)
                    iwait(c0 + 2, sb0, db0, is0)
                    gather(c0 + 2, sb0, rows0, gs0).start()

                return carry

            lax.fori_loop(0, nloops, body, 0)
            plsc.subcore_barrier()
            pltpu.sync_copy(acc.at[pl.ds(sid * rows_pw, rows_pw), :],
                            out_hbm.at[pl.ds(sid * rows_pw, rows_pw), :])

    return segsum, NPAD, calloc


def _bn(x, g, b):
    m = jnp.mean(x, axis=0, keepdims=True)
    v = jnp.mean((x - m) ** 2, axis=0, keepdims=True)
    return (x - m) / jnp.sqrt(v + 1e-5) * g + b


def _emb_body(h_ref, We_ref, be_ref, out_ref):
    out_ref[...] = jnp.dot(h_ref[...], We_ref[...],
                           preferred_element_type=jnp.float32) + be_ref[...]


def _mlp_body(h_ref, parts_ref, W1_ref, b1_ref, g1_ref, bt1_ref,
              W2_ref, b2_ref, ag_ref, ab_ref, lg_ref, lb_ref, out_ref):
    h = h_ref[...]
    n = h.shape[0]
    z = h + parts_ref[:n]
    u = jnp.dot(z, W1_ref[...], preferred_element_type=jnp.float32) + b1_ref[...]
    t = jnp.maximum(_bn(u, g1_ref[...], bt1_ref[...]), 0.0)
    t = jnp.dot(t, W2_ref[...], preferred_element_type=jnp.float32) + b2_ref[...]
    t = jnp.maximum(_bn(t, ag_ref[...], ab_ref[...]), 0.0)
    t = _bn(t, lg_ref[...], lb_ref[...])
    t = jnp.maximum(t, 0.0)
    out_ref[...] = h + t


@functools.lru_cache(maxsize=None)
def _make_dense(N, D):
    emb = pl.pallas_call(
        _emb_body, out_shape=jax.ShapeDtypeStruct((N, D), jnp.float32))
    mlp = pl.pallas_call(
        _mlp_body, out_shape=jax.ShapeDtypeStruct((N, D), jnp.float32))
    return emb, mlp


def kernel(h, edge_index, e, We, be, mW1, mb1, mg1, mbt1, mW2, mb2,
           ag, ab, lg, lb):
    N, D = h.shape
    E = edge_index.shape[1]
    L = mW1.shape[0]
    segsum, NPAD, calloc = _make_segsum(N, E, D)
    emb, mlp = _make_dense(N, D)

    src = edge_index[0].astype(jnp.int32)
    dst = edge_index[1].astype(jnp.int32)
    pad = calloc * _K - E
    if pad:
        src = jnp.concatenate([src, jnp.zeros((pad,), jnp.int32)])
        dst = jnp.concatenate([dst, jnp.full((pad,), N, jnp.int32)])
    src = src.reshape(calloc, _K)
    dst = dst.reshape(calloc, _K)
    zeros = jnp.zeros((NPAD, D), jnp.float32)

    r1 = lambda a: a.reshape(1, D)
    h = emb(h, We, r1(be))
    for l in range(L):
        parts = segsum(h, src, dst, zeros)
        h = mlp(h, parts, mW1[l], r1(mb1[l]), r1(mg1[l]), r1(mbt1[l]),
                mW2[l], r1(mb2[l]), r1(ag[l]), r1(ab[l]), r1(lg[l]), r1(lb[l]))
    return h
